# baseline (device time: 52505 ns/iter reference)
import jax
import jax.numpy as jnp
from jax import lax
from jax.experimental import pallas as pl
from jax.experimental.pallas import tpu as pltpu

N_DEV = 8
SQ = 1024
D = 1024
HQ_LOC = 8
DH = 128
BLK = 64
RC = 256
CH = 64
SCALE = 0.08838834764831843


def kernel(x, Wq, K_ext, V_ext, Wo):
    def body(x_ref, wq_ref, k_hbm, v_hbm, wo_ref, out_ref,
             kland, vland, accb, recvA, recvB,
             copy_sems, send_sems, recv_sems):
        my_pos = lax.axis_index("i")

        hsl = pl.ds(my_pos * HQ_LOC, HQ_LOC)
        kcopy = pltpu.make_async_copy(k_hbm.at[0, :, hsl, :], kland,
                                      copy_sems.at[0])
        vcopy = pltpu.make_async_copy(v_hbm.at[0, :, hsl, :], vland,
                                      copy_sems.at[1])
        kcopy.start()
        vcopy.start()

        barrier_sem = pltpu.get_barrier_semaphore()
        for j in range(N_DEV - 1):
            pl.semaphore_signal(barrier_sem, inc=1,
                                device_id=(lax.rem(my_pos + j + 1, N_DEV),),
                                device_id_type=pl.DeviceIdType.MESH)
        pl.semaphore_wait(barrier_sem, N_DEV - 1)

        wqb = wq_ref[...].astype(jnp.bfloat16)
        wob = wo_ref[...].astype(jnp.bfloat16)
        kcopy.wait()
        vcopy.wait()
        kb = kland[...].astype(jnp.bfloat16)
        vb = vland[...].astype(jnp.bfloat16)

        row_blk = lax.broadcasted_iota(jnp.int32, (RC, RC), 0) // BLK
        col_blk = lax.broadcasted_iota(jnp.int32, (RC, RC), 1) // BLK
        dmask = (col_blk <= row_blk).astype(jnp.float32)

        def compute_chunk(c):
            ext = RC * (c + 1)
            rows = slice(c * RC, (c + 1) * RC)
            xc = x_ref[0, rows, :].astype(jnp.bfloat16)
            qc = jnp.dot(xc, wqb,
                         preferred_element_type=jnp.float32).astype(jnp.bfloat16)
            ctxs = []
            for h in range(HQ_LOC):
                q_h = qc[:, h * DH:(h + 1) * DH]
                s = lax.dot_general(q_h, kb[:ext, h, :],
                                    (((1,), (1,)), ((), ())),
                                    preferred_element_type=jnp.float32) * SCALE
                w = jnp.exp(s)
                wd = (w[:, ext - RC:] * dmask)
                wsum = jnp.sum(wd, axis=1, keepdims=True)
                ctx = jnp.dot(wd.astype(jnp.bfloat16), vb[ext - RC:ext, h, :],
                              preferred_element_type=jnp.float32)
                if ext > RC:
                    wv = w[:, :ext - RC]
                    wsum = wsum + jnp.sum(wv, axis=1, keepdims=True)
                    ctx = ctx + jnp.dot(wv.astype(jnp.bfloat16),
                                        vb[:ext - RC, h, :],
                                        preferred_element_type=jnp.float32)
                ctxs.append((ctx / wsum).astype(jnp.bfloat16))
            ctx_c = jnp.concatenate(ctxs, axis=1)
            accb[rows, :] = jnp.dot(
                ctx_c, wob, preferred_element_type=jnp.float32
            ).astype(jnp.bfloat16)

        def peer(j):
            return lax.rem(my_pos + j + 1, N_DEV)

        def rs_issue(base, sem0, rbuf):
            rds = []
            for j in range(N_DEV - 1):
                p = peer(j)
                rd = pltpu.make_async_remote_copy(
                    src_ref=accb.at[pl.ds(base + p * CH, CH), :],
                    dst_ref=rbuf.at[6 - j],
                    send_sem=send_sems.at[sem0 + j],
                    recv_sem=recv_sems.at[sem0 + 6 - j],
                    device_id=(p,),
                    device_id_type=pl.DeviceIdType.MESH,
                )
                rd.start()
                rds.append(rd)
            return rds

        def rs_reduce(base, rbuf):
            sl = pl.ds(base + my_pos * CH, CH)
            s = accb[sl, :].astype(jnp.float32)
            for k in range(N_DEV - 1):
                s = s + rbuf[k].astype(jnp.float32)
            accb[sl, :] = s.astype(jnp.bfloat16)

        def ag_issue(base, sem0):
            sl = pl.ds(base + my_pos * CH, CH)
            rds = []
            for j in range(N_DEV - 1):
                rd = pltpu.make_async_remote_copy(
                    src_ref=accb.at[sl, :],
                    dst_ref=accb.at[sl, :],
                    send_sem=send_sems.at[sem0 + j],
                    recv_sem=recv_sems.at[sem0 + 6 - j],
                    device_id=(peer(j),),
                    device_id_type=pl.DeviceIdType.MESH,
                )
                rd.start()
                rds.append(rd)
            return rds

        def wait_all(rds):
            for rd in rds:
                rd.wait()

        HALF = SQ // 2

        compute_chunk(0)
        compute_chunk(1)
        a_rs = rs_issue(0, 0, recvA)
        compute_chunk(2)
        wait_all(a_rs)
        rs_reduce(0, recvA)
        a_ag = ag_issue(0, 7)
        compute_chunk(3)
        b_rs = rs_issue(HALF, 14, recvB)
        wait_all(a_ag)
        out_ref[0, :HALF, :] = accb[:HALF, :].astype(jnp.float32)
        wait_all(b_rs)
        rs_reduce(HALF, recvB)
        b_ag = ag_issue(HALF, 21)
        wait_all(b_ag)
        out_ref[0, HALF:, :] = accb[HALF:, :].astype(jnp.float32)

    return pl.pallas_call(
        body,
        out_shape=jax.ShapeDtypeStruct((1, SQ, D), jnp.float32),
        in_specs=[
            pl.BlockSpec(memory_space=pltpu.VMEM),
            pl.BlockSpec(memory_space=pltpu.VMEM),
            pl.BlockSpec(memory_space=pl.ANY),
            pl.BlockSpec(memory_space=pl.ANY),
            pl.BlockSpec(memory_space=pltpu.VMEM),
        ],
        out_specs=pl.BlockSpec(memory_space=pltpu.VMEM),
        scratch_shapes=[
            pltpu.VMEM((SQ, HQ_LOC, DH), jnp.float32),
            pltpu.VMEM((SQ, HQ_LOC, DH), jnp.float32),
            pltpu.VMEM((SQ, D), jnp.bfloat16),
            pltpu.VMEM((N_DEV - 1, CH, D), jnp.bfloat16),
            pltpu.VMEM((N_DEV - 1, CH, D), jnp.bfloat16),
            pltpu.SemaphoreType.DMA((2,)),
            pltpu.SemaphoreType.DMA((28,)),
            pltpu.SemaphoreType.DMA((28,)),
        ],
        compiler_params=pltpu.CompilerParams(collective_id=0),
    )(x, Wq, K_ext, V_ext, Wo)


# device time: 47952 ns/iter; 1.0949x vs baseline; 1.0949x over previous
import jax
import jax.numpy as jnp
from jax import lax
from jax.experimental import pallas as pl
from jax.experimental.pallas import tpu as pltpu

N_DEV = 8
SQ = 1024
D = 1024
HQ_LOC = 8
DH = 128
BLK = 64
RC = 256
CH = 64
SCALE = 0.08838834764831843


def kernel(x, Wq, K_ext, V_ext, Wo):
    def body(x_ref, wq_ref, k_hbm, v_hbm, wo_ref, out_ref,
             kland, vland, accb, recvA, recvM, recvB,
             copy_sems, send_sems, recv_sems):
        my_pos = lax.axis_index("i")

        hsl = pl.ds(my_pos * HQ_LOC, HQ_LOC)
        kcopy = pltpu.make_async_copy(k_hbm.at[0, :, hsl, :], kland,
                                      copy_sems.at[0])
        vcopy = pltpu.make_async_copy(v_hbm.at[0, :, hsl, :], vland,
                                      copy_sems.at[1])
        kcopy.start()
        vcopy.start()

        barrier_sem = pltpu.get_barrier_semaphore()
        for j in range(N_DEV - 1):
            pl.semaphore_signal(barrier_sem, inc=1,
                                device_id=(lax.rem(my_pos + j + 1, N_DEV),),
                                device_id_type=pl.DeviceIdType.MESH)
        pl.semaphore_wait(barrier_sem, N_DEV - 1)

        wqb = wq_ref[...].astype(jnp.bfloat16)
        wob = wo_ref[...].astype(jnp.bfloat16)
        kcopy.wait()
        vcopy.wait()
        kb = kland[...].astype(jnp.bfloat16)
        vb = vland[...].astype(jnp.bfloat16)

        row_blk = lax.broadcasted_iota(jnp.int32, (RC, RC), 0) // BLK
        col_blk = lax.broadcasted_iota(jnp.int32, (RC, RC), 1) // BLK
        dmask = (col_blk <= row_blk).astype(jnp.float32)

        def compute_chunk(c):
            ext = RC * (c + 1)
            rows = slice(c * RC, (c + 1) * RC)
            xc = x_ref[0, rows, :].astype(jnp.bfloat16)
            qc = jnp.dot(xc, wqb,
                         preferred_element_type=jnp.float32).astype(jnp.bfloat16)
            ctxs = []
            for h in range(HQ_LOC):
                q_h = qc[:, h * DH:(h + 1) * DH]
                s = lax.dot_general(q_h, kb[:ext, h, :],
                                    (((1,), (1,)), ((), ())),
                                    preferred_element_type=jnp.float32) * SCALE
                w = jnp.exp(s)
                wd = (w[:, ext - RC:] * dmask)
                wsum = jnp.sum(wd, axis=1, keepdims=True)
                ctx = jnp.dot(wd.astype(jnp.bfloat16), vb[ext - RC:ext, h, :],
                              preferred_element_type=jnp.float32)
                if ext > RC:
                    wv = w[:, :ext - RC]
                    wsum = wsum + jnp.sum(wv, axis=1, keepdims=True)
                    ctx = ctx + jnp.dot(wv.astype(jnp.bfloat16),
                                        vb[:ext - RC, h, :],
                                        preferred_element_type=jnp.float32)
                ctxs.append((ctx / wsum).astype(jnp.bfloat16))
            ctx_c = jnp.concatenate(ctxs, axis=1)
            accb[rows, :] = jnp.dot(
                ctx_c, wob, preferred_element_type=jnp.float32
            ).astype(jnp.bfloat16)

        def peer(j):
            return lax.rem(my_pos + j + 1, N_DEV)

        def rs_issue(base, ch, sem0, rbuf):
            rds = []
            for j in range(N_DEV - 1):
                p = peer(j)
                rd = pltpu.make_async_remote_copy(
                    src_ref=accb.at[pl.ds(base + p * ch, ch), :],
                    dst_ref=rbuf.at[6 - j],
                    send_sem=send_sems.at[sem0 + j],
                    recv_sem=recv_sems.at[sem0 + 6 - j],
                    device_id=(p,),
                    device_id_type=pl.DeviceIdType.MESH,
                )
                rd.start()
                rds.append(rd)
            return rds

        def rs_reduce(base, ch, rbuf):
            sl = pl.ds(base + my_pos * ch, ch)
            s = accb[sl, :].astype(jnp.float32)
            for k in range(N_DEV - 1):
                s = s + rbuf[k].astype(jnp.float32)
            accb[sl, :] = s.astype(jnp.bfloat16)

        def ag_issue(base, ch, sem0):
            sl = pl.ds(base + my_pos * ch, ch)
            rds = []
            for j in range(N_DEV - 1):
                rd = pltpu.make_async_remote_copy(
                    src_ref=accb.at[sl, :],
                    dst_ref=accb.at[sl, :],
                    send_sem=send_sems.at[sem0 + j],
                    recv_sem=recv_sems.at[sem0 + 6 - j],
                    device_id=(peer(j),),
                    device_id_type=pl.DeviceIdType.MESH,
                )
                rd.start()
                rds.append(rd)
            return rds

        def wait_all(rds):
            for rd in rds:
                rd.wait()

        compute_chunk(0)
        compute_chunk(1)
        a_rs = rs_issue(0, 64, 0, recvA)
        compute_chunk(2)
        m_rs = rs_issue(512, 32, 14, recvM)
        wait_all(a_rs)
        rs_reduce(0, 64, recvA)
        a_ag = ag_issue(0, 64, 7)
        compute_chunk(3)
        b_rs = rs_issue(768, 32, 28, recvB)
        wait_all(m_rs)
        rs_reduce(512, 32, recvM)
        m_ag = ag_issue(512, 32, 21)
        wait_all(a_ag)
        out_ref[0, :512, :] = accb[:512, :].astype(jnp.float32)
        wait_all(b_rs)
        rs_reduce(768, 32, recvB)
        b_ag = ag_issue(768, 32, 35)
        wait_all(m_ag)
        out_ref[0, 512:768, :] = accb[512:768, :].astype(jnp.float32)
        wait_all(b_ag)
        out_ref[0, 768:, :] = accb[768:, :].astype(jnp.float32)

    return pl.pallas_call(
        body,
        out_shape=jax.ShapeDtypeStruct((1, SQ, D), jnp.float32),
        in_specs=[
            pl.BlockSpec(memory_space=pltpu.VMEM),
            pl.BlockSpec(memory_space=pltpu.VMEM),
            pl.BlockSpec(memory_space=pl.ANY),
            pl.BlockSpec(memory_space=pl.ANY),
            pl.BlockSpec(memory_space=pltpu.VMEM),
        ],
        out_specs=pl.BlockSpec(memory_space=pltpu.VMEM),
        scratch_shapes=[
            pltpu.VMEM((SQ, HQ_LOC, DH), jnp.float32),
            pltpu.VMEM((SQ, HQ_LOC, DH), jnp.float32),
            pltpu.VMEM((SQ, D), jnp.bfloat16),
            pltpu.VMEM((N_DEV - 1, 64, D), jnp.bfloat16),
            pltpu.VMEM((N_DEV - 1, 32, D), jnp.bfloat16),
            pltpu.VMEM((N_DEV - 1, 32, D), jnp.bfloat16),
            pltpu.SemaphoreType.DMA((2,)),
            pltpu.SemaphoreType.DMA((42,)),
            pltpu.SemaphoreType.DMA((42,)),
        ],
        compiler_params=pltpu.CompilerParams(collective_id=0),
    )(x, Wq, K_ext, V_ext, Wo)


# device time: 47409 ns/iter; 1.1075x vs baseline; 1.0115x over previous
import jax
import jax.numpy as jnp
from jax import lax
from jax.experimental import pallas as pl
from jax.experimental.pallas import tpu as pltpu

N_DEV = 8
SQ = 1024
D = 1024
HQ_LOC = 8
DH = 128
BLK = 64
RC = 256
CH = 64
SCALE = 0.08838834764831843


def kernel(x, Wq, K_ext, V_ext, Wo):
    def body(x_ref, wq_ref, k_hbm, v_hbm, wo_ref, out_ref,
             kland, vland, accb, recvA, recvM, recvB,
             copy_sems, send_sems, recv_sems):
        my_pos = lax.axis_index("i")

        hsl = pl.ds(my_pos * HQ_LOC, HQ_LOC)
        kcopy = pltpu.make_async_copy(k_hbm.at[0, :, hsl, :], kland,
                                      copy_sems.at[0])
        vcopy = pltpu.make_async_copy(v_hbm.at[0, :, hsl, :], vland,
                                      copy_sems.at[1])
        kcopy.start()
        vcopy.start()

        barrier_sem = pltpu.get_barrier_semaphore()
        for j in range(N_DEV - 1):
            pl.semaphore_signal(barrier_sem, inc=1,
                                device_id=(lax.rem(my_pos + j + 1, N_DEV),),
                                device_id_type=pl.DeviceIdType.MESH)

        wqb = (wq_ref[...] * SCALE).astype(jnp.bfloat16)
        wob = wo_ref[...].astype(jnp.bfloat16)
        pl.semaphore_wait(barrier_sem, N_DEV - 1)
        kcopy.wait()
        vcopy.wait()
        kb = kland[...].astype(jnp.bfloat16)
        vb = vland[...].astype(jnp.bfloat16)

        row_blk = lax.broadcasted_iota(jnp.int32, (RC, RC), 0) // BLK
        col_blk = lax.broadcasted_iota(jnp.int32, (RC, RC), 1) // BLK
        dmask = (col_blk <= row_blk).astype(jnp.bfloat16)

        def compute_chunk(c):
            ext = RC * (c + 1)
            rows = slice(c * RC, (c + 1) * RC)
            xc = x_ref[0, rows, :].astype(jnp.bfloat16)
            qc = jnp.dot(xc, wqb,
                         preferred_element_type=jnp.float32).astype(jnp.bfloat16)
            ctxs = []
            for h in range(HQ_LOC):
                q_h = qc[:, h * DH:(h + 1) * DH]
                s = lax.dot_general(q_h, kb[:ext, h, :],
                                    (((1,), (1,)), ((), ())),
                                    preferred_element_type=jnp.float32)
                w = jnp.exp(s.astype(jnp.bfloat16))
                wd = w[:, ext - RC:] * dmask
                wsum = jnp.sum(wd, axis=1, keepdims=True,
                               dtype=jnp.float32)
                ctx = jnp.dot(wd, vb[ext - RC:ext, h, :],
                              preferred_element_type=jnp.float32)
                if ext > RC:
                    wv = w[:, :ext - RC]
                    wsum = wsum + jnp.sum(wv, axis=1, keepdims=True,
                                          dtype=jnp.float32)
                    ctx = ctx + jnp.dot(wv, vb[:ext - RC, h, :],
                                        preferred_element_type=jnp.float32)
                ctxs.append((ctx / wsum).astype(jnp.bfloat16))
            ctx_c = jnp.concatenate(ctxs, axis=1)
            accb[rows, :] = jnp.dot(
                ctx_c, wob, preferred_element_type=jnp.float32
            ).astype(jnp.bfloat16)

        def peer(j):
            return lax.rem(my_pos + j + 1, N_DEV)

        def rs_issue(base, ch, sem0, rbuf):
            rds = []
            for j in range(N_DEV - 1):
                p = peer(j)
                rd = pltpu.make_async_remote_copy(
                    src_ref=accb.at[pl.ds(base + p * ch, ch), :],
                    dst_ref=rbuf.at[6 - j],
                    send_sem=send_sems.at[sem0 + j],
                    recv_sem=recv_sems.at[sem0 + 6 - j],
                    device_id=(p,),
                    device_id_type=pl.DeviceIdType.MESH,
                )
                rd.start()
                rds.append(rd)
            return rds

        def rs_reduce(base, ch, rbuf):
            sl = pl.ds(base + my_pos * ch, ch)
            s = accb[sl, :].astype(jnp.float32)
            for k in range(N_DEV - 1):
                s = s + rbuf[k].astype(jnp.float32)
            accb[sl, :] = s.astype(jnp.bfloat16)

        def ag_issue(base, ch, sem0):
            sl = pl.ds(base + my_pos * ch, ch)
            rds = []
            for j in range(N_DEV - 1):
                rd = pltpu.make_async_remote_copy(
                    src_ref=accb.at[sl, :],
                    dst_ref=accb.at[sl, :],
                    send_sem=send_sems.at[sem0 + j],
                    recv_sem=recv_sems.at[sem0 + 6 - j],
                    device_id=(peer(j),),
                    device_id_type=pl.DeviceIdType.MESH,
                )
                rd.start()
                rds.append(rd)
            return rds

        def wait_all(rds):
            for rd in rds:
                rd.wait()

        compute_chunk(0)
        compute_chunk(1)
        a_rs = rs_issue(0, 64, 0, recvA)
        compute_chunk(2)
        m_rs = rs_issue(512, 32, 14, recvM)
        wait_all(a_rs)
        rs_reduce(0, 64, recvA)
        a_ag = ag_issue(0, 64, 7)
        compute_chunk(3)
        b_rs = rs_issue(768, 32, 28, recvB)
        wait_all(m_rs)
        rs_reduce(512, 32, recvM)
        m_ag = ag_issue(512, 32, 21)
        wait_all(a_ag)
        out_ref[0, :512, :] = accb[:512, :].astype(jnp.float32)
        wait_all(b_rs)
        rs_reduce(768, 32, recvB)
        b_ag = ag_issue(768, 32, 35)
        wait_all(m_ag)
        out_ref[0, 512:768, :] = accb[512:768, :].astype(jnp.float32)
        wait_all(b_ag)
        out_ref[0, 768:, :] = accb[768:, :].astype(jnp.float32)

    return pl.pallas_call(
        body,
        out_shape=jax.ShapeDtypeStruct((1, SQ, D), jnp.float32),
        in_specs=[
            pl.BlockSpec(memory_space=pltpu.VMEM),
            pl.BlockSpec(memory_space=pltpu.VMEM),
            pl.BlockSpec(memory_space=pl.ANY),
            pl.BlockSpec(memory_space=pl.ANY),
            pl.BlockSpec(memory_space=pltpu.VMEM),
        ],
        out_specs=pl.BlockSpec(memory_space=pltpu.VMEM),
        scratch_shapes=[
            pltpu.VMEM((SQ, HQ_LOC, DH), jnp.float32),
            pltpu.VMEM((SQ, HQ_LOC, DH), jnp.float32),
            pltpu.VMEM((SQ, D), jnp.bfloat16),
            pltpu.VMEM((N_DEV - 1, 64, D), jnp.bfloat16),
            pltpu.VMEM((N_DEV - 1, 32, D), jnp.bfloat16),
            pltpu.VMEM((N_DEV - 1, 32, D), jnp.bfloat16),
            pltpu.SemaphoreType.DMA((2,)),
            pltpu.SemaphoreType.DMA((42,)),
            pltpu.SemaphoreType.DMA((42,)),
        ],
        compiler_params=pltpu.CompilerParams(collective_id=0),
    )(x, Wq, K_ext, V_ext, Wo)


# device time: 45126 ns/iter; 1.1635x vs baseline; 1.0506x over previous
import jax
import jax.numpy as jnp
from jax import lax
from jax.experimental import pallas as pl
from jax.experimental.pallas import tpu as pltpu

N_DEV = 8
SQ = 1024
D = 1024
HQ_LOC = 8
DH = 128
BLK = 64
RC = 256
CH = 64
SCALE = 0.08838834764831843


def kernel(x, Wq, K_ext, V_ext, Wo):
    def body(x_ref, wq_ref, k_hbm, v_hbm, wo_ref, out_ref,
             kland, vland, accb, recv0, recv1, recv2, recv3,
             copy_sems, send_sems, recv_sems):
        my_pos = lax.axis_index("i")

        hsl = pl.ds(my_pos * HQ_LOC, HQ_LOC)
        kcopy = pltpu.make_async_copy(k_hbm.at[0, :, hsl, :], kland,
                                      copy_sems.at[0])
        vcopy = pltpu.make_async_copy(v_hbm.at[0, :, hsl, :], vland,
                                      copy_sems.at[1])
        kcopy.start()
        vcopy.start()

        barrier_sem = pltpu.get_barrier_semaphore()
        for j in range(N_DEV - 1):
            pl.semaphore_signal(barrier_sem, inc=1,
                                device_id=(lax.rem(my_pos + j + 1, N_DEV),),
                                device_id_type=pl.DeviceIdType.MESH)

        wqb = (wq_ref[...] * SCALE).astype(jnp.bfloat16)
        wob = wo_ref[...].astype(jnp.bfloat16)
        pl.semaphore_wait(barrier_sem, N_DEV - 1)
        kcopy.wait()
        vcopy.wait()
        kb = kland[...].astype(jnp.bfloat16)
        vb = vland[...].astype(jnp.bfloat16)

        row_blk = lax.broadcasted_iota(jnp.int32, (RC, RC), 0) // BLK
        col_blk = lax.broadcasted_iota(jnp.int32, (RC, RC), 1) // BLK
        dmask = (col_blk <= row_blk).astype(jnp.bfloat16)

        def compute_chunk(c):
            ext = RC * (c + 1)
            rows = slice(c * RC, (c + 1) * RC)
            xc = x_ref[0, rows, :].astype(jnp.bfloat16)
            qc = jnp.dot(xc, wqb,
                         preferred_element_type=jnp.float32).astype(jnp.bfloat16)
            ctxs = []
            for h in range(HQ_LOC):
                q_h = qc[:, h * DH:(h + 1) * DH]
                s = lax.dot_general(q_h, kb[:ext, h, :],
                                    (((1,), (1,)), ((), ())),
                                    preferred_element_type=jnp.float32)
                w = jnp.exp(s.astype(jnp.bfloat16))
                wd = w[:, ext - RC:] * dmask
                wsum = jnp.sum(wd, axis=1, keepdims=True,
                               dtype=jnp.float32)
                ctx = jnp.dot(wd, vb[ext - RC:ext, h, :],
                              preferred_element_type=jnp.float32)
                if ext > RC:
                    wv = w[:, :ext - RC]
                    wsum = wsum + jnp.sum(wv, axis=1, keepdims=True,
                                          dtype=jnp.float32)
                    ctx = ctx + jnp.dot(wv, vb[:ext - RC, h, :],
                                        preferred_element_type=jnp.float32)
                ctxs.append((ctx / wsum).astype(jnp.bfloat16))
            ctx_c = jnp.concatenate(ctxs, axis=1)
            accb[rows, :] = jnp.dot(
                ctx_c, wob, preferred_element_type=jnp.float32
            ).astype(jnp.bfloat16)

        def peer(j):
            return lax.rem(my_pos + j + 1, N_DEV)

        def rs_issue(base, ch, sem0, rbuf):
            rds = []
            for j in range(N_DEV - 1):
                p = peer(j)
                rd = pltpu.make_async_remote_copy(
                    src_ref=accb.at[pl.ds(base + p * ch, ch), :],
                    dst_ref=rbuf.at[6 - j],
                    send_sem=send_sems.at[sem0 + j],
                    recv_sem=recv_sems.at[sem0 + 6 - j],
                    device_id=(p,),
                    device_id_type=pl.DeviceIdType.MESH,
                )
                rd.start()
                rds.append(rd)
            return rds

        def rs_reduce(base, ch, rbuf):
            sl = pl.ds(base + my_pos * ch, ch)
            s = accb[sl, :].astype(jnp.float32)
            for k in range(N_DEV - 1):
                s = s + rbuf[k].astype(jnp.float32)
            accb[sl, :] = s.astype(jnp.bfloat16)

        def ag_issue(base, ch, sem0):
            sl = pl.ds(base + my_pos * ch, ch)
            rds = []
            for j in range(N_DEV - 1):
                rd = pltpu.make_async_remote_copy(
                    src_ref=accb.at[sl, :],
                    dst_ref=accb.at[sl, :],
                    send_sem=send_sems.at[sem0 + j],
                    recv_sem=recv_sems.at[sem0 + 6 - j],
                    device_id=(peer(j),),
                    device_id_type=pl.DeviceIdType.MESH,
                )
                rd.start()
                rds.append(rd)
            return rds

        def wait_all(rds):
            for rd in rds:
                rd.wait()

        compute_chunk(0)
        g0 = rs_issue(0, 32, 0, recv0)
        compute_chunk(1)
        g1 = rs_issue(256, 32, 14, recv1)
        compute_chunk(2)
        g2 = rs_issue(512, 32, 28, recv2)
        wait_all(g0)
        rs_reduce(0, 32, recv0)
        ag0 = ag_issue(0, 32, 7)
        wait_all(g1)
        rs_reduce(256, 32, recv1)
        ag1 = ag_issue(256, 32, 21)
        compute_chunk(3)
        g3 = rs_issue(768, 32, 42, recv3)
        wait_all(ag0)
        out_ref[0, :256, :] = accb[:256, :].astype(jnp.float32)
        wait_all(g2)
        rs_reduce(512, 32, recv2)
        ag2 = ag_issue(512, 32, 35)
        wait_all(ag1)
        out_ref[0, 256:512, :] = accb[256:512, :].astype(jnp.float32)
        wait_all(g3)
        rs_reduce(768, 32, recv3)
        ag3 = ag_issue(768, 32, 49)
        wait_all(ag2)
        out_ref[0, 512:768, :] = accb[512:768, :].astype(jnp.float32)
        wait_all(ag3)
        out_ref[0, 768:, :] = accb[768:, :].astype(jnp.float32)

    return pl.pallas_call(
        body,
        out_shape=jax.ShapeDtypeStruct((1, SQ, D), jnp.float32),
        in_specs=[
            pl.BlockSpec(memory_space=pltpu.VMEM),
            pl.BlockSpec(memory_space=pltpu.VMEM),
            pl.BlockSpec(memory_space=pl.ANY),
            pl.BlockSpec(memory_space=pl.ANY),
            pl.BlockSpec(memory_space=pltpu.VMEM),
        ],
        out_specs=pl.BlockSpec(memory_space=pltpu.VMEM),
        scratch_shapes=[
            pltpu.VMEM((SQ, HQ_LOC, DH), jnp.float32),
            pltpu.VMEM((SQ, HQ_LOC, DH), jnp.float32),
            pltpu.VMEM((SQ, D), jnp.bfloat16),
            pltpu.VMEM((N_DEV - 1, 32, D), jnp.bfloat16),
            pltpu.VMEM((N_DEV - 1, 32, D), jnp.bfloat16),
            pltpu.VMEM((N_DEV - 1, 32, D), jnp.bfloat16),
            pltpu.VMEM((N_DEV - 1, 32, D), jnp.bfloat16),
            pltpu.SemaphoreType.DMA((2,)),
            pltpu.SemaphoreType.DMA((56,)),
            pltpu.SemaphoreType.DMA((56,)),
        ],
        compiler_params=pltpu.CompilerParams(collective_id=0),
    )(x, Wq, K_ext, V_ext, Wo)
